# baseline (speedup 1.0000x reference)
import jax
import jax.numpy as jnp
from jax import lax
from jax.experimental import pallas as pl
from jax.experimental.pallas import tpu as pltpu
from jax.experimental.pallas import tpu_sc as plsc

NC = 2
NS = 16
NW = NC * NS
LANES = 16
CHUNK = 128


def _make_body(BPW, NCH, D):
    nvec = D // LANES

    def body(x_hbm, idx_hbm, center_hbm, out_hbm,
             idx_v, x_v, rows_v, acc_v, sem0, sem1):
        wid = lax.axis_index("s") * NC + lax.axis_index("c")
        base = wid * BPW
        sems = [sem0, sem1]

        pltpu.sync_copy(idx_hbm.at[pl.ds(wid * NCH, NCH)], idx_v)

        def fire(c):
            return pltpu.async_copy(
                center_hbm.at[idx_v.at[c]], rows_v.at[c % 2], sems[c % 2])

        copies = {0: fire(0), 1: fire(1)}
        pltpu.sync_copy(x_hbm.at[pl.ds(base, BPW)], x_v)

        zero = jnp.zeros((LANES,), jnp.float32)
        accs = (zero,) * nvec

        for c in range(NCH):
            copies[c].wait()
            buf = c % 2

            def item_body(i, a, _c=c, _buf=buf):
                new = []
                for j in range(nvec):
                    xv = x_v[_c * CHUNK + i, pl.ds(j * LANES, LANES)]
                    cv = rows_v[_buf, i, pl.ds(j * LANES, LANES)]
                    d = xv - cv
                    new.append(a[j] + d * d)
                return tuple(new)

            accs = lax.fori_loop(0, CHUNK, item_body, accs)
            if c + 2 < NCH:
                copies[c + 2] = fire(c + 2)

        total = accs[0]
        for j in range(1, nvec):
            total = total + accs[j]
        acc_v[...] = total
        pltpu.sync_copy(acc_v, out_hbm.at[wid])

    return body


def _pad_transpose_block(in_ref, out_ref):
    t = in_ref[...].T
    out_ref[...] = jnp.concatenate(
        [t, jnp.zeros((128, 64), jnp.float32)], axis=1)


def _pad_transpose(a_t, n_rows):
    grid = (n_rows + 127) // 128
    return pl.pallas_call(
        _pad_transpose_block,
        grid=(grid,),
        in_specs=[pl.BlockSpec((64, 128), lambda g: (0, g))],
        out_specs=pl.BlockSpec((128, 128), lambda g: (g, 0)),
        out_shape=jax.ShapeDtypeStruct((n_rows, 128), jnp.float32),
    )(a_t)


@jax.jit
def kernel(inputs, labels, center):
    B, D = inputs.shape
    BPW = B // NW
    NCH = BPW // CHUNK

    idx2d = labels.astype(jnp.int32).reshape(NW * NCH, CHUNK)
    x_p = _pad_transpose(inputs.T, B)
    center_p = _pad_transpose(center.T, center.shape[0])

    mesh = plsc.VectorSubcoreMesh(core_axis_name="c", subcore_axis_name="s")
    body = _make_body(BPW, NCH, D)

    partials = pl.kernel(
        body,
        out_type=jax.ShapeDtypeStruct((NW, LANES), jnp.float32),
        mesh=mesh,
        scratch_types=[
            pltpu.VMEM((NCH, CHUNK), jnp.int32),
            pltpu.VMEM((BPW, 128), jnp.float32),
            pltpu.VMEM((2, CHUNK, 128), jnp.float32),
            pltpu.VMEM((LANES,), jnp.float32),
            pltpu.SemaphoreType.DMA,
            pltpu.SemaphoreType.DMA,
        ],
        compiler_params=pltpu.CompilerParams(use_tc_tiling_on_sc=False),
    )(x_p, idx2d, center_p)

    return jnp.sum(partials) * (1.0 / (B * D))


# speedup vs baseline: 8.1160x; 8.1160x over previous
import jax
import jax.numpy as jnp
from jax import lax
from jax.experimental import pallas as pl
from jax.experimental.pallas import tpu as pltpu
from jax.experimental.pallas import tpu_sc as plsc

NC = 2
NS = 16
NW = NC * NS
LANES = 16
CHUNK = 128


def _make_body(BPW, NCH, D):
    nvec = D // LANES

    def body(x_hbm, idx_hbm, center_hbm, out_hbm,
             idx_v, x_v, rows_v, acc_v, sem0, sem1):
        wid = lax.axis_index("s") * NC + lax.axis_index("c")
        base = wid * BPW
        sems = [sem0, sem1]

        pltpu.sync_copy(idx_hbm.at[pl.ds(wid * NCH, NCH)], idx_v)

        def fire(c):
            return pltpu.async_copy(
                center_hbm.at[idx_v.at[c]], rows_v.at[c % 2], sems[c % 2])

        copies = {0: fire(0), 1: fire(1)}
        pltpu.sync_copy(x_hbm.at[pl.ds(base, BPW)], x_v)

        zero = jnp.zeros((LANES,), jnp.float32)
        accs = (zero,) * nvec

        for c in range(NCH):
            copies[c].wait()
            buf = c % 2

            def item_body(i, a, _c=c, _buf=buf):
                new = []
                for j in range(nvec):
                    xv = x_v[_c * CHUNK + i, pl.ds(j * LANES, LANES)]
                    cv = rows_v[_buf, i, pl.ds(j * LANES, LANES)]
                    d = xv - cv
                    new.append(a[j] + d * d)
                return tuple(new)

            accs = lax.fori_loop(0, CHUNK, item_body, accs)
            if c + 2 < NCH:
                copies[c + 2] = fire(c + 2)

        total = accs[0]
        for j in range(1, nvec):
            total = total + accs[j]
        acc_v[...] = total
        pltpu.sync_copy(acc_v, out_hbm.at[wid])

    return body


def _make_pad_transpose_block(bk):
    def blockfn(in_ref, out_ref):
        t = in_ref[...].T
        out_ref[...] = jnp.concatenate(
            [t, jnp.zeros((bk, 64), jnp.float32)], axis=1)
    return blockfn


def _pad_transpose(a_t, n_rows, bk):
    grid = (n_rows + bk - 1) // bk
    return pl.pallas_call(
        _make_pad_transpose_block(bk),
        grid=(grid,),
        in_specs=[pl.BlockSpec((64, bk), lambda g: (0, g))],
        out_specs=pl.BlockSpec((bk, 128), lambda g: (g, 0)),
        out_shape=jax.ShapeDtypeStruct((n_rows, 128), jnp.float32),
    )(a_t)


@jax.jit
def kernel(inputs, labels, center):
    B, D = inputs.shape
    BPW = B // NW
    NCH = BPW // CHUNK

    idx2d = labels.astype(jnp.int32).reshape(NW * NCH, CHUNK)
    x_p = _pad_transpose(inputs.T, B, 8192)
    center_p = _pad_transpose(center.T, center.shape[0], 8192)

    mesh = plsc.VectorSubcoreMesh(core_axis_name="c", subcore_axis_name="s")
    body = _make_body(BPW, NCH, D)

    partials = pl.kernel(
        body,
        out_type=jax.ShapeDtypeStruct((NW, LANES), jnp.float32),
        mesh=mesh,
        scratch_types=[
            pltpu.VMEM((NCH, CHUNK), jnp.int32),
            pltpu.VMEM((BPW, 128), jnp.float32),
            pltpu.VMEM((2, CHUNK, 128), jnp.float32),
            pltpu.VMEM((LANES,), jnp.float32),
            pltpu.SemaphoreType.DMA,
            pltpu.SemaphoreType.DMA,
        ],
        compiler_params=pltpu.CompilerParams(use_tc_tiling_on_sc=False),
    )(x_p, idx2d, center_p)

    return jnp.sum(partials) * (1.0 / (B * D))
